# baseline (speedup 1.0000x reference)
import functools
import jax
import jax.numpy as jnp
from jax import lax
from jax.experimental import pallas as pl
from jax.experimental.pallas import tpu as pltpu
from jax.experimental.pallas import tpu_sc as plsc

D = 64
SCALE = 8.0
NW = 32
VOCAB_SIZE = 1_000_000
BLK = 256
NBLK = (VOCAB_SIZE - 64) // BLK
TAIL = VOCAB_SIZE - NBLK * BLK
K1_RING = 4
K2_U = 4
GB_RING = 4
TB_RING = 2
PF = 4


def _iota16():
    return lax.iota(jnp.int32, 16)


@functools.partial(
    pl.kernel,
    out_type=jax.ShapeDtypeStruct((VOCAB_SIZE // 2, 2 * D), jnp.float32),
    mesh=plsc.VectorSubcoreMesh(core_axis_name="c", subcore_axis_name="s"),
    scratch_types=[
        [pltpu.VMEM((D, BLK), jnp.float32) for _ in range(K1_RING)],
        [pltpu.VMEM((BLK // 2, 2 * D), jnp.float32) for _ in range(2)],
        [pltpu.SemaphoreType.DMA for _ in range(K1_RING)],
        [pltpu.SemaphoreType.DMA for _ in range(2)],
    ],
    compiler_params=pltpu.CompilerParams(
        use_tc_tiling_on_sc=True, needs_layout_passes=False
    ),
)
def _transpose_lut(lutT_hbm, tail_hbm, scr_hbm, bufs, obufs, ssems, osems):
    wid = lax.axis_index("s") * 2 + lax.axis_index("c")
    per = NBLK // NW
    rem = NBLK - per * NW
    cnt = jnp.where(wid < rem, per + 1, per)
    lo = wid * per + jnp.minimum(wid, rem)
    riota = [_iota16() + 16 * m for m in range(4)]

    def fire_stage(u, b):
        for band in range(8):
            pltpu.async_copy(
                lutT_hbm.at[pl.ds(8 * band, 8), pl.ds(u * BLK, BLK)],
                bufs[b].at[pl.ds(8 * band, 8)],
                ssems[b],
            )

    def wait_stage(u, b):
        for band in range(8):
            pltpu.make_async_copy(
                lutT_hbm.at[pl.ds(8 * band, 8), pl.ds(u * BLK, BLK)],
                bufs[b].at[pl.ds(8 * band, 8)],
                ssems[b],
            ).wait()

    for p in range(K1_RING - 1):
        fire_stage(lo + p, p)

    n_slots = -(-(per + 1) // K1_RING) * K1_RING

    def body(i, carry):
        for b in range(K1_RING):
            u = i * K1_RING + b

            @pl.when(u < cnt)
            def _():
                blk = lo + u

                @pl.when(u + (K1_RING - 1) < cnt)
                def _():
                    fire_stage(blk + (K1_RING - 1), (b + K1_RING - 1) % K1_RING)

                wait_stage(blk, b)
                ob = b % 2

                @pl.when(u >= 2)
                def _():
                    pltpu.make_async_copy(
                        obufs[ob], scr_hbm.at[pl.ds(0, BLK // 2)], osems[ob]
                    ).wait()

                obuf = obufs[ob]
                buf = bufs[b]

                @plsc.parallel_loop(0, BLK // 2, unroll=4)
                def trans_body(q):
                    c0 = riota[0] * 0 + 2 * q
                    for k in range(8):
                        v = plsc.load_gather(buf, [riota[k % 4], c0 + (k // 4)])
                        obuf[q, pl.ds(16 * k, 16)] = v * SCALE
                pltpu.async_copy(
                    obuf, scr_hbm.at[pl.ds(blk * (BLK // 2), BLK // 2)], osems[ob]
                )
        return carry

    lax.fori_loop(0, n_slots // K1_RING, body, 0)
    for ob in range(2):
        pltpu.make_async_copy(
            obufs[ob], scr_hbm.at[pl.ds(0, BLK // 2)], osems[ob]
        ).wait()

    @pl.when(wid == NW - 1)
    def _():
        tb = obufs[0].at[pl.ds(0, TAIL // 2)]
        pltpu.sync_copy(tail_hbm, tb)

        def tail_body(q, c2):
            for k in range(8):
                sl = pl.ds(16 * k, 16)
                tb[q, sl] = tb[q, sl] * SCALE
            return c2

        lax.fori_loop(0, TAIL // 2, tail_body, 0)
        pltpu.sync_copy(tb, scr_hbm.at[pl.ds(NBLK * (BLK // 2), TAIL // 2)])


def _make_gather(b_total, seq):
    rows_per_worker = b_total // NW

    @functools.partial(
        pl.kernel,
        out_type=jax.ShapeDtypeStruct((seq, 8, b_total // 128, 8 * 128), jnp.float32),
        mesh=plsc.VectorSubcoreMesh(core_axis_name="c", subcore_axis_name="s"),
        scratch_types=[
            pltpu.VMEM((rows_per_worker // 2, seq), jnp.int32),
            pltpu.VMEM((seq, rows_per_worker), jnp.int32),
            [pltpu.VMEM((rows_per_worker, D), jnp.float32) for _ in range(GB_RING)],
            [pltpu.VMEM((8, 8 * 128), jnp.float32) for _ in range(TB_RING)],
            [pltpu.SemaphoreType.DMA for _ in range(GB_RING)],
            [pltpu.SemaphoreType.DMA for _ in range(TB_RING)],
        ],
        compiler_params=pltpu.CompilerParams(
            use_tc_tiling_on_sc=False, needs_layout_passes=False
        ),
    )
    def k(scr_hbm, x_hbm, out_hbm, xbuf, idxT, gbufs, tbufs, gsems, osems):
        wid = lax.axis_index("s") * 2 + lax.axis_index("c")
        row0 = wid * rows_per_worker
        riota = [_iota16() + 16 * m for m in range(8)]

        for h in range(2):
            pltpu.sync_copy(
                x_hbm.at[pl.ds(row0 + 64 * h, rows_per_worker // 2)], xbuf
            )

            @plsc.parallel_loop(0, seq, unroll=4)
            def xt_body(q):
                c0 = riota[0] * 0 + q
                for k in range(4):
                    v = plsc.load_gather(xbuf, [riota[k], c0])
                    idxT[q, pl.ds(64 * h + 16 * k, 16)] = v

        def fire_gather(c, b):
            pltpu.async_copy(scr_hbm.at[idxT.at[c]], gbufs[b], gsems[b])

        def drain_gather(c, b):
            pltpu.make_async_copy(
                scr_hbm.at[idxT.at[c]], gbufs[b], gsems[b]
            ).wait()

        for b in range(PF):
            fire_gather(b, b)

        def body(i, carry):
            for b in range(K2_U):
                c = i * K2_U + b
                tb = b % TB_RING

                drain_gather(c, b % GB_RING)
                gbuf = gbufs[b % GB_RING]
                tbuf = tbufs[tb]

                @pl.when(c >= TB_RING)
                def _():
                    for dblk in range(8):
                        pltpu.make_async_copy(
                            tbufs[tb].at[dblk], out_hbm.at[0, dblk, wid], osems[tb]
                        ).wait()

                @plsc.parallel_loop(0, 8, unroll=2)
                def ext_body(dblk):
                    for d_in in range(8):
                        dcol = riota[0] * 0 + (dblk * 8 + d_in)
                        for g in range(8):
                            v = plsc.load_gather(gbuf, [riota[g], dcol])
                            tbuf[dblk, pl.ds(128 * d_in + 16 * g, 16)] = v
                for dblk in range(8):
                    pltpu.async_copy(
                        tbuf.at[dblk], out_hbm.at[c, dblk, wid], osems[tb]
                    )

                @pl.when(c + PF <= seq - 1)
                def _():
                    fire_gather(c + PF, (b + PF) % GB_RING)
            return carry

        lax.fori_loop(0, seq // K2_U, body, 0)
        for tb in range(TB_RING):
            for dblk in range(8):
                pltpu.make_async_copy(
                    tbufs[tb].at[dblk], out_hbm.at[0, dblk, wid], osems[tb]
                ).wait()

    return k


@jax.jit
def _embed(x, lut):
    b_total, seq = x.shape
    lutT = jnp.swapaxes(lut, 0, 1)
    tail = lut[NBLK * BLK :].reshape(TAIL // 2, 2 * D)
    scr = _transpose_lut(lutT, tail)
    scr_rows = scr.reshape(VOCAB_SIZE, D)
    out4 = _make_gather(b_total, seq)(scr_rows, x)
    out5 = out4.reshape(seq, 8, b_total // 128, 8, 128)
    return jnp.transpose(out5, (2, 4, 0, 1, 3)).reshape(b_total, seq, D)


def kernel(x, lut):
    b0, b1 = x.shape
    assert b0 % NW == 0 and b1 % K2_U == 0
    return _embed(x.astype(jnp.int32), lut)


# speedup vs baseline: 1.3714x; 1.3714x over previous
import functools
import jax
import jax.numpy as jnp
from jax import lax
from jax.experimental import pallas as pl
from jax.experimental.pallas import tpu as pltpu
from jax.experimental.pallas import tpu_sc as plsc
from jax.experimental.layout import Format, Layout

D_MODEL = 64
SCALE = 8.0
NUM_WORKERS = 32
SPLIT = 96
NBUF = 4


def _gather_scale(lut, x):
    b_total, seq = x.shape
    rows_per_worker = b_total // NUM_WORKERS
    mesh = plsc.VectorSubcoreMesh(core_axis_name="c", subcore_axis_name="s")

    @functools.partial(
        pl.kernel,
        out_type=jax.ShapeDtypeStruct((b_total, seq, D_MODEL), jnp.float32),
        mesh=mesh,
        scratch_types=[
            pltpu.VMEM((rows_per_worker, seq), jnp.int32),
            [pltpu.VMEM((seq, D_MODEL), jnp.float32) for _ in range(NBUF)],
            [pltpu.SemaphoreType.DMA for _ in range(NBUF)],
            [pltpu.SemaphoreType.DMA for _ in range(NBUF)],
        ],
        compiler_params=pltpu.CompilerParams(use_tc_tiling_on_sc=False),
    )
    def k(lut_hbm, x_hbm, out_hbm, idx_all, bufs, gsems, osems):
        wid = lax.axis_index("s") * 2 + lax.axis_index("c")
        row0 = wid * rows_per_worker

        pltpu.sync_copy(x_hbm.at[pl.ds(row0, rows_per_worker)], idx_all)

        def fire_gather(c, b):
            pltpu.async_copy(
                lut_hbm.at[idx_all.at[c, pl.ds(0, SPLIT)]],
                bufs[b].at[pl.ds(0, SPLIT)],
                gsems[b],
            )
            pltpu.async_copy(
                lut_hbm.at[idx_all.at[c, pl.ds(SPLIT, seq - SPLIT)]],
                bufs[b].at[pl.ds(SPLIT, seq - SPLIT)],
                gsems[b],
            )

        def drain_gather(c, b):
            pltpu.make_async_copy(
                lut_hbm.at[idx_all.at[c, pl.ds(0, SPLIT)]],
                bufs[b].at[pl.ds(0, SPLIT)],
                gsems[b],
            ).wait()
            pltpu.make_async_copy(
                lut_hbm.at[idx_all.at[c, pl.ds(SPLIT, seq - SPLIT)]],
                bufs[b].at[pl.ds(SPLIT, seq - SPLIT)],
                gsems[b],
            ).wait()

        PF = NBUF - 2

        for b in range(PF):
            fire_gather(b, b)

        def body(i, carry):
            for b in range(NBUF):
                c = i * NBUF + b
                b_pre = (b + PF) % NBUF

                @pl.when(c + PF <= rows_per_worker - 1)
                def _():
                    @pl.when(c >= PF)
                    def _():
                        pltpu.make_async_copy(
                            bufs[b_pre],
                            out_hbm.at[row0],
                            osems[b_pre],
                        ).wait()

                    fire_gather(c + PF, b_pre)

                drain_gather(c, b)

                buf = bufs[b]

                def scale_body(r, c2):
                    for rr in range(4):
                        for j in range(D_MODEL // 16):
                            sl = pl.ds(j * 16, 16)
                            buf[r * 4 + rr, sl] = buf[r * 4 + rr, sl] * SCALE
                    return c2

                lax.fori_loop(0, seq // 4, scale_body, 0, unroll=2)

                pltpu.async_copy(buf, out_hbm.at[row0 + c], osems[b])
            return carry

        lax.fori_loop(0, rows_per_worker // NBUF, body, 0)

        for b in range(NBUF):
            pltpu.make_async_copy(
                bufs[b], out_hbm.at[row0], osems[b]
            ).wait()

    return k(lut, x)


_jit_cache = {}


def _jitted_for(dev):
    if dev not in _jit_cache:
        fmt = Format(
            Layout(major_to_minor=(0, 1, 2)),
            jax.sharding.SingleDeviceSharding(dev),
        )
        _jit_cache[dev] = jax.jit(_gather_scale, out_shardings=fmt)
    return _jit_cache[dev]


def kernel(x, lut):
    b0, b1 = x.shape
    assert b0 % NUM_WORKERS == 0 and (b0 // NUM_WORKERS) % NBUF == 0
    assert b1 % 8 == 0 and SPLIT % 8 == 0
    try:
        dev = next(iter(lut.devices()))
        fn = _jitted_for(dev)
    except Exception:
        if "_plain" not in _jit_cache:
            _jit_cache["_plain"] = jax.jit(_gather_scale)
        fn = _jit_cache["_plain"]
    return fn(lut, x.astype(jnp.int32))


# speedup vs baseline: 1.3720x; 1.0004x over previous
import functools
import jax
import jax.numpy as jnp
from jax import lax
from jax.experimental import pallas as pl
from jax.experimental.pallas import tpu as pltpu
from jax.experimental.pallas import tpu_sc as plsc

D_MODEL = 64
SCALE = 8.0
NUM_WORKERS = 32
SPLIT = 96
NBUF = 4


@jax.jit
def _gather_scale(lut, x):
    b_total, seq = x.shape
    rows_per_worker = b_total // NUM_WORKERS
    mesh = plsc.VectorSubcoreMesh(core_axis_name="c", subcore_axis_name="s")

    @functools.partial(
        pl.kernel,
        out_type=jax.ShapeDtypeStruct((b_total, seq, D_MODEL), jnp.float32),
        mesh=mesh,
        scratch_types=[
            pltpu.VMEM((rows_per_worker, seq), jnp.int32),
            [pltpu.VMEM((seq, D_MODEL), jnp.float32) for _ in range(NBUF)],
            [pltpu.SemaphoreType.DMA for _ in range(NBUF)],
            [pltpu.SemaphoreType.DMA for _ in range(NBUF)],
        ],
        compiler_params=pltpu.CompilerParams(use_tc_tiling_on_sc=False),
    )
    def k(lut_hbm, x_hbm, out_hbm, idx_all, bufs, gsems, osems):
        wid = lax.axis_index("s") * 2 + lax.axis_index("c")
        row0 = wid * rows_per_worker

        pltpu.sync_copy(x_hbm.at[pl.ds(row0, rows_per_worker)], idx_all)

        def fire_gather(c, b):
            pltpu.async_copy(
                lut_hbm.at[idx_all.at[c, pl.ds(0, SPLIT)]],
                bufs[b].at[pl.ds(0, SPLIT)],
                gsems[b],
            )
            pltpu.async_copy(
                lut_hbm.at[idx_all.at[c, pl.ds(SPLIT, seq - SPLIT)]],
                bufs[b].at[pl.ds(SPLIT, seq - SPLIT)],
                gsems[b],
            )

        def drain_gather(c, b):
            pltpu.make_async_copy(
                lut_hbm.at[idx_all.at[c, pl.ds(0, SPLIT)]],
                bufs[b].at[pl.ds(0, SPLIT)],
                gsems[b],
            ).wait()
            pltpu.make_async_copy(
                lut_hbm.at[idx_all.at[c, pl.ds(SPLIT, seq - SPLIT)]],
                bufs[b].at[pl.ds(SPLIT, seq - SPLIT)],
                gsems[b],
            ).wait()

        PF = NBUF - 2

        for b in range(PF):
            fire_gather(b, b)

        def body(i, carry):
            for b in range(NBUF):
                c = i * NBUF + b
                b_pre = (b + PF) % NBUF

                @pl.when(c + PF <= rows_per_worker - 1)
                def _():
                    @pl.when(c >= PF)
                    def _():
                        pltpu.make_async_copy(
                            bufs[b_pre],
                            out_hbm.at[row0],
                            osems[b_pre],
                        ).wait()

                    fire_gather(c + PF, b_pre)

                drain_gather(c, b)

                buf = bufs[b]

                def scale_body(r, c2):
                    for rr in range(4):
                        for j in range(D_MODEL // 16):
                            sl = pl.ds(j * 16, 16)
                            buf[r * 4 + rr, sl] = buf[r * 4 + rr, sl] * SCALE
                    return c2

                lax.fori_loop(0, seq // 4, scale_body, 0, unroll=2)

                pltpu.async_copy(buf, out_hbm.at[row0 + c], osems[b])
            return carry

        lax.fori_loop(0, rows_per_worker // NBUF, body, 0)

        for b in range(NBUF):
            pltpu.make_async_copy(
                bufs[b], out_hbm.at[row0], osems[b]
            ).wait()

    return k(lut, x)


def kernel(x, lut):
    b0, b1 = x.shape
    assert b0 % NUM_WORKERS == 0 and (b0 // NUM_WORKERS) % NBUF == 0
    assert b1 % 8 == 0 and SPLIT % 8 == 0
    return _gather_scale(lut, x.astype(jnp.int32))
